# baseline (speedup 1.0000x reference)
import jax
import jax.numpy as jnp
from jax import lax
from jax.experimental import pallas as pl
from jax.experimental.pallas import tpu as pltpu
from jax.experimental.pallas import tpu_sc as plsc

_B, _N, _C = 4, 8192, 1024
_NC, _NS, _L = 2, 16, 16
_NW = _NC * _NS
_N_SC = 3072
_ROWS_PER_W = _N_SC // _NW
_R = 16
_RB = _ROWS_PER_W // _R
_NBUF = 3
_UNROLL = 8
_STEPS = _RB * _B
_BN = 1024


def _sc_body(tok_hbm, emb_hbm, out_hbm, emb_v,
             tok0, tok1, tok2, isem0, isem1, isem2, osem0, osem1, osem2):
    tok_bufs = (tok0, tok1, tok2)
    in_sems = (isem0, isem1, isem2)
    out_sems = (osem0, osem1, osem2)
    wid = lax.axis_index("s") * _NC + lax.axis_index("c")
    row_base = wid * _ROWS_PER_W

    def rows(step):
        rb, b = step // _B, step % _B
        return b, row_base + rb * _R

    def start_in(step):
        p = step % _NBUF
        b, r0 = rows(step)
        pltpu.async_copy(
            tok_hbm.at[b, pl.ds(r0, _R), :], tok_bufs[p], in_sems[p])

    def wait_in(step):
        p = step % _NBUF
        b, r0 = rows(step)
        pltpu.make_async_copy(
            tok_hbm.at[b, pl.ds(r0, _R), :], tok_bufs[p], in_sems[p]).wait()

    def start_out(step):
        p = step % _NBUF
        b, r0 = rows(step)
        pltpu.async_copy(
            tok_bufs[p], out_hbm.at[b, pl.ds(r0, _R), :], out_sems[p])

    def wait_out(step):
        p = step % _NBUF
        b, r0 = rows(step)
        pltpu.make_async_copy(
            tok_bufs[p], out_hbm.at[b, pl.ds(r0, _R), :], out_sems[p]).wait()

    for s in range(_NBUF - 1):
        start_in(s)

    for s in range(_STEPS):
        p = s % _NBUF
        rb, b = s // _B, s % _B
        if b == 0:
            pltpu.sync_copy(
                emb_hbm.at[pl.ds(row_base + rb * _R, _R), :], emb_v)
        wait_in(s)

        @plsc.parallel_loop(0, _R * _C, _L, unroll=_UNROLL)
        def _add(i):
            r = i >> 10
            c = pl.multiple_of(i & (_C - 1), _L)
            tok_bufs[p][r, pl.ds(c, _L)] += emb_v[r, pl.ds(c, _L)]

        if s >= 1:
            wait_out(s - 1)
        if s + _NBUF - 1 < _STEPS:
            start_in(s + _NBUF - 1)
        start_out(s)

    wait_out(_STEPS - 1)


def _sc_add(tokens, emb):
    mesh = plsc.VectorSubcoreMesh(core_axis_name="c", subcore_axis_name="s")
    return pl.kernel(
        _sc_body,
        out_type=jax.ShapeDtypeStruct((_B, _N_SC, _C), jnp.float32),
        mesh=mesh,
        scratch_types=(
            [pltpu.VMEM((_R, _C), jnp.float32)]
            + [pltpu.VMEM((_R, _C), jnp.float32) for _ in range(_NBUF)]
            + [pltpu.SemaphoreType.DMA for _ in range(2 * _NBUF)]
        ),
    )(tokens, emb)


def _tc_add_body(tok_ref, emb_ref, out_ref):
    out_ref[...] = tok_ref[...] + emb_ref[...]


def _tc_add(tokens, emb):
    base = _N_SC // _BN
    grid = ((_N - _N_SC) // _BN, _B)
    return pl.pallas_call(
        _tc_add_body,
        grid=grid,
        in_specs=[
            pl.BlockSpec((1, _BN, _C), lambda i, j: (j, i + base, 0)),
            pl.BlockSpec((_BN, _C), lambda i, j: (i + base, 0)),
        ],
        out_specs=pl.BlockSpec((1, _BN, _C), lambda i, j: (j, i + base, 0)),
        out_shape=jax.ShapeDtypeStruct((_B, _N, _C), jnp.float32),
    )(tokens, emb)


def kernel(tokens, emb):
    tc_full = _tc_add(tokens, emb)
    sc_part = _sc_add(tokens, emb)
    return lax.dynamic_update_slice(tc_full, sc_part, (0, 0, 0))


# speedup vs baseline: 1.1104x; 1.1104x over previous
import jax
import jax.numpy as jnp
from jax import lax
from jax.experimental import pallas as pl
from jax.experimental.pallas import tpu as pltpu
from jax.experimental.pallas import tpu_sc as plsc

_B, _N, _C = 4, 8192, 1024
_NC, _NS, _L = 2, 16, 16
_NW = _NC * _NS
_ROWS_PER_W = _N // _NW
_R = 16
_RB = _ROWS_PER_W // _R
_NBUF = 4
_UNROLL = 8
_STEPS = _RB * _B


def _sc_body(tok_hbm, emb_hbm, out_hbm, emb0, emb1,
             tok0, tok1, tok2, tok3,
             esem0, esem1, isem0, isem1, isem2, isem3,
             osem0, osem1, osem2, osem3):
    emb_bufs = (emb0, emb1)
    emb_sems = (esem0, esem1)
    tok_bufs = (tok0, tok1, tok2, tok3)
    in_sems = (isem0, isem1, isem2, isem3)
    out_sems = (osem0, osem1, osem2, osem3)
    wid = lax.axis_index("s") * _NC + lax.axis_index("c")
    row_base = wid * _ROWS_PER_W

    def rows(step):
        rb, b = step // _B, step % _B
        return b, row_base + rb * _R

    def start_in(step):
        p = step % _NBUF
        b, r0 = rows(step)
        pltpu.async_copy(
            tok_hbm.at[b, pl.ds(r0, _R), :], tok_bufs[p], in_sems[p])

    def wait_in(step):
        p = step % _NBUF
        b, r0 = rows(step)
        pltpu.make_async_copy(
            tok_hbm.at[b, pl.ds(r0, _R), :], tok_bufs[p], in_sems[p]).wait()

    def start_out(step):
        p = step % _NBUF
        b, r0 = rows(step)
        pltpu.async_copy(
            tok_bufs[p], out_hbm.at[b, pl.ds(r0, _R), :], out_sems[p])

    def wait_out(step):
        p = step % _NBUF
        b, r0 = rows(step)
        pltpu.make_async_copy(
            tok_bufs[p], out_hbm.at[b, pl.ds(r0, _R), :], out_sems[p]).wait()

    def start_emb(rb):
        pltpu.async_copy(
            emb_hbm.at[pl.ds(row_base + rb * _R, _R), :],
            emb_bufs[rb % 2], emb_sems[rb % 2])

    def wait_emb(rb):
        pltpu.make_async_copy(
            emb_hbm.at[pl.ds(row_base + rb * _R, _R), :],
            emb_bufs[rb % 2], emb_sems[rb % 2]).wait()

    start_emb(0)
    for s in range(_NBUF - 1):
        start_in(s)

    for s in range(_STEPS):
        p = s % _NBUF
        rb, b = s // _B, s % _B
        if b == 0:
            wait_emb(rb)
        if b == 1 and rb + 1 < _RB:
            start_emb(rb + 1)
        wait_in(s)
        emb_v = emb_bufs[rb % 2]

        @plsc.parallel_loop(0, _R * _C, _L, unroll=_UNROLL)
        def _add(i):
            r = i >> 10
            c = pl.multiple_of(i & (_C - 1), _L)
            plsc.addupdate(tok_bufs[p].at[r, pl.ds(c, _L)],
                           emb_v[r, pl.ds(c, _L)])

        if s >= 1:
            wait_out(s - 1)
        if s + _NBUF - 1 < _STEPS:
            start_in(s + _NBUF - 1)
        start_out(s)

    wait_out(_STEPS - 1)


@jax.jit
def _sc_add(tokens, emb):
    mesh = plsc.VectorSubcoreMesh(core_axis_name="c", subcore_axis_name="s")
    return pl.kernel(
        _sc_body,
        out_type=jax.ShapeDtypeStruct((_B, _N, _C), jnp.float32),
        mesh=mesh,
        scratch_types=(
            [pltpu.VMEM((_R, _C), jnp.float32) for _ in range(2)]
            + [pltpu.VMEM((_R, _C), jnp.float32) for _ in range(_NBUF)]
            + [pltpu.SemaphoreType.DMA for _ in range(2 + 2 * _NBUF)]
        ),
    )(tokens, emb)


def kernel(tokens, emb):
    return _sc_add(tokens, emb)
